# Initial kernel scaffold; baseline (speedup 1.0000x reference)
import functools

import jax
import jax.numpy as jnp
from jax import lax
from jax.experimental import pallas as pl
from jax.experimental.pallas import tpu as pltpu
from jax.experimental.pallas import tpu_sc as plsc

N = 10000
E = 160000
D_IN = 256
HID = 32
OUT = 2

NC = 2
NS = 16
NW = NC * NS

NP = 10240
C = 128
CHUNKS_W = 40
EW = C * CHUNKS_W
EP = EW * NW
ROWS_S = NP // NS

_sc_mesh = plsc.VectorSubcoreMesh(
    core_axis_name="c", subcore_axis_name="s", num_cores=NC, num_subcores=NS
)


@functools.partial(
    pl.kernel,
    out_type=jax.ShapeDtypeStruct((NC, NP), jnp.float32),
    mesh=_sc_mesh,
    scratch_types=[
        pltpu.VMEM((CHUNKS_W, C), jnp.int32),
        pltpu.VMEM((C,), jnp.float32),
        pltpu.MemorySpace.VMEM_SHARED((NP,), jnp.float32),
    ],
)
def _sc_degree(dst_hbm, zeros_hbm, out_hbm, dst_v, ones_v, deg_sh):
    c = lax.axis_index("c")
    s = lax.axis_index("s")
    w = c * NS + s
    pltpu.sync_copy(zeros_hbm.at[pl.ds(s * ROWS_S, ROWS_S)],
                    deg_sh.at[pl.ds(s * ROWS_S, ROWS_S)])
    pltpu.sync_copy(dst_hbm.at[pl.ds(w * CHUNKS_W, CHUNKS_W)], dst_v)
    for i in range(C // 16):
        ones_v[pl.ds(i * 16, 16)] = jnp.ones((16,), jnp.float32)
    plsc.subcore_barrier()

    @pl.loop(0, CHUNKS_W)
    def _(j):
        pltpu.sync_copy(ones_v, deg_sh.at[dst_v.at[j]], add=True)

    plsc.subcore_barrier()
    pltpu.sync_copy(deg_sh.at[pl.ds(s * ROWS_S, ROWS_S)],
                    out_hbm.at[c, pl.ds(s * ROWS_S, ROWS_S)])


@functools.partial(
    pl.kernel,
    out_type=jax.ShapeDtypeStruct((NC, NP, HID), jnp.float32),
    mesh=_sc_mesh,
    scratch_types=[
        pltpu.VMEM((CHUNKS_W, C), jnp.int32),
        pltpu.VMEM((CHUNKS_W, C), jnp.int32),
        pltpu.VMEM((C, HID), jnp.float32),
        pltpu.MemorySpace.VMEM_SHARED((NP, HID), jnp.float32),
        pltpu.MemorySpace.VMEM_SHARED((NP, HID), jnp.float32),
        pltpu.SemaphoreType.DMA,
    ],
)
def _sc_aggregate(g_hbm, src_hbm, dst_hbm, zeros_hbm, out_hbm,
                  src_v, dst_v, rows_v, g_sh, agg_sh, sem):
    c = lax.axis_index("c")
    s = lax.axis_index("s")
    w = c * NS + s
    row0 = s * ROWS_S
    pltpu.sync_copy(g_hbm.at[pl.ds(row0, ROWS_S)], g_sh.at[pl.ds(row0, ROWS_S)])
    pltpu.sync_copy(zeros_hbm.at[pl.ds(row0, ROWS_S)],
                    agg_sh.at[pl.ds(row0, ROWS_S)])
    pltpu.sync_copy(src_hbm.at[pl.ds(w * CHUNKS_W, CHUNKS_W)], src_v)
    pltpu.sync_copy(dst_hbm.at[pl.ds(w * CHUNKS_W, CHUNKS_W)], dst_v)
    plsc.subcore_barrier()

    @pl.loop(0, CHUNKS_W)
    def _(j):
        pltpu.async_copy(g_sh.at[src_v.at[j]], rows_v, sem).wait()
        pltpu.sync_copy(rows_v, agg_sh.at[dst_v.at[j]], add=True)

    plsc.subcore_barrier()
    pltpu.sync_copy(agg_sh.at[pl.ds(row0, ROWS_S)],
                    out_hbm.at[c, pl.ds(row0, ROWS_S)])


_BLK = 1024


def _norm_of(deg_ref):
    return lax.rsqrt(deg_ref[0] + deg_ref[1] + 1.0)


def _k1_body(x_ref, w1_ref, deg_ref, g1_ref):
    norm = _norm_of(deg_ref)
    h = jnp.dot(x_ref[...], w1_ref[...], preferred_element_type=jnp.float32)
    g1_ref[...] = h * norm


def _k3_body(agg_ref, g1_ref, deg_ref, b1_ref, w2_ref, g2_ref):
    norm = _norm_of(deg_ref)
    h = jnp.maximum(
        norm * (agg_ref[0] + agg_ref[1] + g1_ref[...]) + b1_ref[...], 0.0)
    g2_ref[...] = jnp.dot(h, w2_ref[...],
                          preferred_element_type=jnp.float32) * norm


def _k4_body(agg_ref, g2_ref, deg_ref, b2_ref, w3_ref, b3_ref, out_ref):
    norm = _norm_of(deg_ref)
    h = jnp.maximum(
        norm * (agg_ref[0] + agg_ref[1] + g2_ref[...]) + b2_ref[...], 0.0)
    logits = jnp.dot(h, w3_ref[...],
                     preferred_element_type=jnp.float32) + b3_ref[...]
    m = jnp.max(logits, axis=-1, keepdims=True)
    e = jnp.exp(logits - m)
    out_ref[...] = e / jnp.sum(e, axis=-1, keepdims=True)


def _tc_k1(x, W1, deg):
    return pl.pallas_call(
        _k1_body,
        grid=(NP // _BLK,),
        in_specs=[
            pl.BlockSpec((_BLK, D_IN), lambda i: (i, 0)),
            pl.BlockSpec((D_IN, HID), lambda i: (0, 0)),
            pl.BlockSpec((NC, _BLK, 1), lambda i: (0, i, 0)),
        ],
        out_specs=pl.BlockSpec((_BLK, HID), lambda i: (i, 0)),
        out_shape=jax.ShapeDtypeStruct((NP, HID), jnp.float32),
    )(x, W1, deg)


def _tc_k3(agg, g1, deg, b1, W2):
    return pl.pallas_call(
        _k3_body,
        grid=(NP // _BLK,),
        in_specs=[
            pl.BlockSpec((NC, _BLK, HID), lambda i: (0, i, 0)),
            pl.BlockSpec((_BLK, HID), lambda i: (i, 0)),
            pl.BlockSpec((NC, _BLK, 1), lambda i: (0, i, 0)),
            pl.BlockSpec((1, HID), lambda i: (0, 0)),
            pl.BlockSpec((HID, HID), lambda i: (0, 0)),
        ],
        out_specs=pl.BlockSpec((_BLK, HID), lambda i: (i, 0)),
        out_shape=jax.ShapeDtypeStruct((NP, HID), jnp.float32),
    )(agg, g1, deg, b1, W2)


def _tc_k4(agg, g2, deg, b2, W3, b3):
    return pl.pallas_call(
        _k4_body,
        grid=(NP // _BLK,),
        in_specs=[
            pl.BlockSpec((NC, _BLK, HID), lambda i: (0, i, 0)),
            pl.BlockSpec((_BLK, HID), lambda i: (i, 0)),
            pl.BlockSpec((NC, _BLK, 1), lambda i: (0, i, 0)),
            pl.BlockSpec((1, HID), lambda i: (0, 0)),
            pl.BlockSpec((HID, OUT), lambda i: (0, 0)),
            pl.BlockSpec((1, OUT), lambda i: (0, 0)),
        ],
        out_specs=pl.BlockSpec((_BLK, OUT), lambda i: (i, 0)),
        out_shape=jax.ShapeDtypeStruct((NP, OUT), jnp.float32),
    )(agg, g2, deg, b2, W3, b3)


@jax.jit
def kernel(x, edge_index, W1, b1, W2, b2, W3, b3):
    x_p = jnp.zeros((NP, D_IN), jnp.float32).at[:N].set(x)
    pad = jnp.full((2, EP - E), N, jnp.int32)
    ei = jnp.concatenate([edge_index, pad], axis=1)
    src2 = ei[0].reshape(NW * CHUNKS_W, C)
    dst2 = ei[1].reshape(NW * CHUNKS_W, C)
    zeros_col = jnp.zeros((NP,), jnp.float32)
    zeros_feat = jnp.zeros((NP, HID), jnp.float32)

    deg = _sc_degree(dst2, zeros_col)
    deg3 = deg.reshape(NC, NP, 1)
    g1 = _tc_k1(x_p, W1, deg3)
    agg1 = _sc_aggregate(g1, src2, dst2, zeros_feat)
    g2 = _tc_k3(agg1, g1, deg3, b1.reshape(1, HID), W2)
    agg2 = _sc_aggregate(g2, src2, dst2, zeros_feat)
    probs = _tc_k4(agg2, g2, deg3, b2.reshape(1, HID), W3, b3.reshape(1, OUT))
    return probs[:N]


# speedup vs baseline: 24.2377x; 24.2377x over previous
import functools

import jax
import jax.numpy as jnp
from jax import lax
from jax.experimental import pallas as pl
from jax.experimental.pallas import tpu as pltpu
from jax.experimental.pallas import tpu_sc as plsc

N = 10000
E = 160000
D_IN = 256
HID = 32
OUT = 2

NC = 2
NS = 16
NW = NC * NS

NP = 10240
C = 128
CHUNKS_W = 40
EW = C * CHUNKS_W
EP = EW * NW
ROWS_S = NP // NS

_sc_mesh = plsc.VectorSubcoreMesh(
    core_axis_name="c", subcore_axis_name="s", num_cores=NC, num_subcores=NS
)
_sc_params = pltpu.CompilerParams(use_tc_tiling_on_sc=False)


@functools.partial(
    pl.kernel,
    out_type=jax.ShapeDtypeStruct((NC, NP), jnp.float32),
    mesh=_sc_mesh,
    scratch_types=[
        pltpu.VMEM((CHUNKS_W, C), jnp.int32),
        pltpu.VMEM((C,), jnp.float32),
        pltpu.MemorySpace.VMEM_SHARED((NP,), jnp.float32),
    ],
    compiler_params=_sc_params,
)
def _sc_degree(dst_hbm, zeros_hbm, out_hbm, dst_v, ones_v, deg_sh):
    c = lax.axis_index("c")
    s = lax.axis_index("s")
    w = c * NS + s
    pltpu.sync_copy(zeros_hbm.at[pl.ds(s * ROWS_S, ROWS_S)],
                    deg_sh.at[pl.ds(s * ROWS_S, ROWS_S)])
    pltpu.sync_copy(dst_hbm.at[pl.ds(w * CHUNKS_W, CHUNKS_W)], dst_v)
    for i in range(C // 16):
        ones_v[pl.ds(i * 16, 16)] = jnp.ones((16,), jnp.float32)
    plsc.subcore_barrier()

    @pl.loop(0, CHUNKS_W)
    def _(j):
        pltpu.sync_copy(ones_v, deg_sh.at[dst_v.at[j]], add=True)

    plsc.subcore_barrier()
    pltpu.sync_copy(deg_sh.at[pl.ds(s * ROWS_S, ROWS_S)],
                    out_hbm.at[c, pl.ds(s * ROWS_S, ROWS_S)])


@functools.partial(
    pl.kernel,
    out_type=jax.ShapeDtypeStruct((NC, NP, HID), jnp.float32),
    mesh=_sc_mesh,
    scratch_types=[
        pltpu.VMEM((CHUNKS_W, C), jnp.int32),
        pltpu.VMEM((CHUNKS_W, C), jnp.int32),
        pltpu.VMEM((C, HID), jnp.float32),
        pltpu.MemorySpace.VMEM_SHARED((NP, HID), jnp.float32),
        pltpu.MemorySpace.VMEM_SHARED((NP, HID), jnp.float32),
        pltpu.SemaphoreType.DMA,
    ],
    compiler_params=_sc_params,
)
def _sc_aggregate(g_hbm, src_hbm, dst_hbm, zeros_hbm, out_hbm,
                  src_v, dst_v, rows_v, g_sh, agg_sh, sem):
    c = lax.axis_index("c")
    s = lax.axis_index("s")
    w = c * NS + s
    row0 = s * ROWS_S
    pltpu.sync_copy(g_hbm.at[pl.ds(row0, ROWS_S)], g_sh.at[pl.ds(row0, ROWS_S)])
    pltpu.sync_copy(zeros_hbm.at[pl.ds(row0, ROWS_S)],
                    agg_sh.at[pl.ds(row0, ROWS_S)])
    pltpu.sync_copy(src_hbm.at[pl.ds(w * CHUNKS_W, CHUNKS_W)], src_v)
    pltpu.sync_copy(dst_hbm.at[pl.ds(w * CHUNKS_W, CHUNKS_W)], dst_v)
    plsc.subcore_barrier()

    @pl.loop(0, CHUNKS_W)
    def _(j):
        pltpu.async_copy(g_sh.at[src_v.at[j]], rows_v, sem).wait()
        pltpu.sync_copy(rows_v, agg_sh.at[dst_v.at[j]], add=True)

    plsc.subcore_barrier()
    pltpu.sync_copy(agg_sh.at[pl.ds(row0, ROWS_S)],
                    out_hbm.at[c, pl.ds(row0, ROWS_S)])


_BLK = 1024


def _norm_of(deg_ref):
    return lax.rsqrt(deg_ref[0] + deg_ref[1] + 1.0)


def _k1_body(x_ref, w1_ref, deg_ref, g1_ref):
    norm = _norm_of(deg_ref)
    h = jnp.dot(x_ref[...], w1_ref[...], preferred_element_type=jnp.float32)
    g1_ref[...] = h * norm


def _k3_body(agg_ref, g1_ref, deg_ref, b1_ref, w2_ref, g2_ref):
    norm = _norm_of(deg_ref)
    h = jnp.maximum(
        norm * (agg_ref[0] + agg_ref[1] + g1_ref[...]) + b1_ref[...], 0.0)
    g2_ref[...] = jnp.dot(h, w2_ref[...],
                          preferred_element_type=jnp.float32) * norm


def _k4_body(agg_ref, g2_ref, deg_ref, b2_ref, w3_ref, b3_ref, out_ref):
    norm = _norm_of(deg_ref)
    h = jnp.maximum(
        norm * (agg_ref[0] + agg_ref[1] + g2_ref[...]) + b2_ref[...], 0.0)
    logits = jnp.dot(h, w3_ref[...],
                     preferred_element_type=jnp.float32) + b3_ref[...]
    m = jnp.max(logits, axis=-1, keepdims=True)
    e = jnp.exp(logits - m)
    out_ref[...] = e / jnp.sum(e, axis=-1, keepdims=True)


def _tc_k1(x, W1, deg):
    return pl.pallas_call(
        _k1_body,
        grid=(NP // _BLK,),
        in_specs=[
            pl.BlockSpec((_BLK, D_IN), lambda i: (i, 0)),
            pl.BlockSpec((D_IN, HID), lambda i: (0, 0)),
            pl.BlockSpec((NC, _BLK, 1), lambda i: (0, i, 0)),
        ],
        out_specs=pl.BlockSpec((_BLK, HID), lambda i: (i, 0)),
        out_shape=jax.ShapeDtypeStruct((NP, HID), jnp.float32),
    )(x, W1, deg)


def _tc_k3(agg, g1, deg, b1, W2):
    return pl.pallas_call(
        _k3_body,
        grid=(NP // _BLK,),
        in_specs=[
            pl.BlockSpec((NC, _BLK, HID), lambda i: (0, i, 0)),
            pl.BlockSpec((_BLK, HID), lambda i: (i, 0)),
            pl.BlockSpec((NC, _BLK, 1), lambda i: (0, i, 0)),
            pl.BlockSpec((1, HID), lambda i: (0, 0)),
            pl.BlockSpec((HID, HID), lambda i: (0, 0)),
        ],
        out_specs=pl.BlockSpec((_BLK, HID), lambda i: (i, 0)),
        out_shape=jax.ShapeDtypeStruct((NP, HID), jnp.float32),
    )(agg, g1, deg, b1, W2)


def _tc_k4(agg, g2, deg, b2, W3, b3):
    return pl.pallas_call(
        _k4_body,
        grid=(NP // _BLK,),
        in_specs=[
            pl.BlockSpec((NC, _BLK, HID), lambda i: (0, i, 0)),
            pl.BlockSpec((_BLK, HID), lambda i: (i, 0)),
            pl.BlockSpec((NC, _BLK, 1), lambda i: (0, i, 0)),
            pl.BlockSpec((1, HID), lambda i: (0, 0)),
            pl.BlockSpec((HID, OUT), lambda i: (0, 0)),
            pl.BlockSpec((1, OUT), lambda i: (0, 0)),
        ],
        out_specs=pl.BlockSpec((_BLK, OUT), lambda i: (i, 0)),
        out_shape=jax.ShapeDtypeStruct((NP, OUT), jnp.float32),
    )(agg, g2, deg, b2, W3, b3)


@jax.jit
def kernel(x, edge_index, W1, b1, W2, b2, W3, b3):
    x_p = jnp.zeros((NP, D_IN), jnp.float32).at[:N].set(x)
    pad = jnp.full((2, EP - E), N, jnp.int32)
    ei = jnp.concatenate([edge_index, pad], axis=1)
    src2 = ei[0].reshape(NW * CHUNKS_W, C)
    dst2 = ei[1].reshape(NW * CHUNKS_W, C)
    zeros_col = jnp.zeros((NP,), jnp.float32)
    zeros_feat = jnp.zeros((NP, HID), jnp.float32)

    deg = _sc_degree(dst2, zeros_col)
    deg3 = deg.reshape(NC, NP, 1)
    g1 = _tc_k1(x_p, W1, deg3)
    agg1 = _sc_aggregate(g1, src2, dst2, zeros_feat)
    g2 = _tc_k3(agg1, g1, deg3, b1.reshape(1, HID), W2)
    agg2 = _sc_aggregate(g2, src2, dst2, zeros_feat)
    probs = _tc_k4(agg2, g2, deg3, b2.reshape(1, HID), W3, b3.reshape(1, OUT))
    return probs[:N]


# speedup vs baseline: 25.2864x; 1.0433x over previous
import functools

import jax
import jax.numpy as jnp
from jax import lax
from jax.experimental import pallas as pl
from jax.experimental.pallas import tpu as pltpu
from jax.experimental.pallas import tpu_sc as plsc

N = 10000
E = 160000
D_IN = 256
HID = 32
OUT = 2

NC = 2
NS = 16
NW = NC * NS

NP = 10240
C = 128
CHUNKS_W = 40
EW = C * CHUNKS_W
EP = EW * NW
ROWS_S = NP // NS

_sc_mesh = plsc.VectorSubcoreMesh(
    core_axis_name="c", subcore_axis_name="s", num_cores=NC, num_subcores=NS
)
_sc_params = pltpu.CompilerParams(use_tc_tiling_on_sc=False)


@functools.partial(
    pl.kernel,
    out_type=jax.ShapeDtypeStruct((NC, NP), jnp.float32),
    mesh=_sc_mesh,
    scratch_types=[
        pltpu.VMEM((CHUNKS_W, C), jnp.int32),
        pltpu.VMEM((C,), jnp.float32),
        pltpu.MemorySpace.VMEM_SHARED((NP,), jnp.float32),
    ],
    compiler_params=_sc_params,
)
def _sc_degree(dst_hbm, zeros_hbm, out_hbm, dst_v, ones_v, deg_sh):
    c = lax.axis_index("c")
    s = lax.axis_index("s")
    w = c * NS + s
    pltpu.sync_copy(zeros_hbm.at[pl.ds(s * ROWS_S, ROWS_S)],
                    deg_sh.at[pl.ds(s * ROWS_S, ROWS_S)])
    pltpu.sync_copy(dst_hbm.at[pl.ds(w * CHUNKS_W, CHUNKS_W)], dst_v)
    for i in range(C // 16):
        ones_v[pl.ds(i * 16, 16)] = jnp.ones((16,), jnp.float32)
    plsc.subcore_barrier()

    @pl.loop(0, CHUNKS_W)
    def _(j):
        pltpu.sync_copy(ones_v, deg_sh.at[dst_v.at[j]], add=True)

    plsc.subcore_barrier()
    pltpu.sync_copy(deg_sh.at[pl.ds(s * ROWS_S, ROWS_S)],
                    out_hbm.at[c, pl.ds(s * ROWS_S, ROWS_S)])


@functools.partial(
    pl.kernel,
    out_type=jax.ShapeDtypeStruct((NC, NP, HID), jnp.float32),
    mesh=_sc_mesh,
    scratch_types=[
        pltpu.VMEM((CHUNKS_W, C), jnp.int32),
        pltpu.VMEM((CHUNKS_W, C), jnp.int32),
        pltpu.VMEM((C, HID), jnp.float32),
        pltpu.MemorySpace.VMEM_SHARED((NP, HID), jnp.float32),
        pltpu.MemorySpace.VMEM_SHARED((NP, HID), jnp.float32),
        pltpu.SemaphoreType.DMA,
    ],
    compiler_params=_sc_params,
)
def _sc_aggregate(g_hbm, src_hbm, dst_hbm, zeros_hbm, out_hbm,
                  src_v, dst_v, rows_v, g_sh, agg_sh, sem):
    c = lax.axis_index("c")
    s = lax.axis_index("s")
    w = c * NS + s
    row0 = s * ROWS_S
    pltpu.sync_copy(g_hbm.at[pl.ds(row0, ROWS_S)], g_sh.at[pl.ds(row0, ROWS_S)])
    pltpu.sync_copy(zeros_hbm.at[pl.ds(row0, ROWS_S)],
                    agg_sh.at[pl.ds(row0, ROWS_S)])
    pltpu.sync_copy(src_hbm.at[pl.ds(w * CHUNKS_W, CHUNKS_W)], src_v)
    pltpu.sync_copy(dst_hbm.at[pl.ds(w * CHUNKS_W, CHUNKS_W)], dst_v)
    plsc.subcore_barrier()

    @pl.loop(0, CHUNKS_W)
    def _(j):
        pltpu.async_copy(g_sh.at[src_v.at[j]], rows_v, sem).wait()
        pltpu.sync_copy(rows_v, agg_sh.at[dst_v.at[j]], add=True)

    plsc.subcore_barrier()
    pltpu.sync_copy(agg_sh.at[pl.ds(row0, ROWS_S)],
                    out_hbm.at[c, pl.ds(row0, ROWS_S)])


_BLK = 2560


def _norm_of(deg_ref):
    return lax.rsqrt(deg_ref[0] + deg_ref[1] + 1.0)


def _k1_body(x_ref, w1_ref, deg_ref, g1_ref):
    norm = _norm_of(deg_ref)
    h = jnp.dot(x_ref[...], w1_ref[...], preferred_element_type=jnp.float32)
    g1_ref[...] = h * norm


def _k3_body(agg_ref, g1_ref, deg_ref, b1_ref, w2_ref, g2_ref):
    norm = _norm_of(deg_ref)
    h = jnp.maximum(
        norm * (agg_ref[0] + agg_ref[1] + g1_ref[...]) + b1_ref[...], 0.0)
    g2_ref[...] = jnp.dot(h, w2_ref[...],
                          preferred_element_type=jnp.float32) * norm


def _k4_body(agg_ref, g2_ref, deg_ref, b2_ref, w3_ref, b3_ref, out_ref):
    norm = _norm_of(deg_ref)
    h = jnp.maximum(
        norm * (agg_ref[0] + agg_ref[1] + g2_ref[...]) + b2_ref[...], 0.0)
    logits = jnp.dot(h, w3_ref[...],
                     preferred_element_type=jnp.float32) + b3_ref[...]
    m = jnp.max(logits, axis=-1, keepdims=True)
    e = jnp.exp(logits - m)
    out_ref[...] = e / jnp.sum(e, axis=-1, keepdims=True)


def _tc_k1(x, W1, deg):
    return pl.pallas_call(
        _k1_body,
        grid=(NP // _BLK,),
        in_specs=[
            pl.BlockSpec((_BLK, D_IN), lambda i: (i, 0)),
            pl.BlockSpec((D_IN, HID), lambda i: (0, 0)),
            pl.BlockSpec((NC, _BLK, 1), lambda i: (0, i, 0)),
        ],
        out_specs=pl.BlockSpec((_BLK, HID), lambda i: (i, 0)),
        out_shape=jax.ShapeDtypeStruct((NP, HID), jnp.float32),
    )(x, W1, deg)


def _tc_k3(agg, g1, deg, b1, W2):
    return pl.pallas_call(
        _k3_body,
        grid=(NP // _BLK,),
        in_specs=[
            pl.BlockSpec((NC, _BLK, HID), lambda i: (0, i, 0)),
            pl.BlockSpec((_BLK, HID), lambda i: (i, 0)),
            pl.BlockSpec((NC, _BLK, 1), lambda i: (0, i, 0)),
            pl.BlockSpec((1, HID), lambda i: (0, 0)),
            pl.BlockSpec((HID, HID), lambda i: (0, 0)),
        ],
        out_specs=pl.BlockSpec((_BLK, HID), lambda i: (i, 0)),
        out_shape=jax.ShapeDtypeStruct((NP, HID), jnp.float32),
    )(agg, g1, deg, b1, W2)


def _tc_k4(agg, g2, deg, b2, W3, b3):
    return pl.pallas_call(
        _k4_body,
        grid=(NP // _BLK,),
        in_specs=[
            pl.BlockSpec((NC, _BLK, HID), lambda i: (0, i, 0)),
            pl.BlockSpec((_BLK, HID), lambda i: (i, 0)),
            pl.BlockSpec((NC, _BLK, 1), lambda i: (0, i, 0)),
            pl.BlockSpec((1, HID), lambda i: (0, 0)),
            pl.BlockSpec((HID, OUT), lambda i: (0, 0)),
            pl.BlockSpec((1, OUT), lambda i: (0, 0)),
        ],
        out_specs=pl.BlockSpec((_BLK, OUT), lambda i: (i, 0)),
        out_shape=jax.ShapeDtypeStruct((NP, OUT), jnp.float32),
    )(agg, g2, deg, b2, W3, b3)


@jax.jit
def kernel(x, edge_index, W1, b1, W2, b2, W3, b3):
    x_p = jnp.zeros((NP, D_IN), jnp.float32).at[:N].set(x)
    pad = jnp.full((2, EP - E), N, jnp.int32)
    ei = jnp.concatenate([edge_index, pad], axis=1)
    src2 = ei[0].reshape(NW * CHUNKS_W, C)
    dst2 = ei[1].reshape(NW * CHUNKS_W, C)
    zeros_col = jnp.zeros((NP,), jnp.float32)
    zeros_feat = jnp.zeros((NP, HID), jnp.float32)

    deg = _sc_degree(dst2, zeros_col)
    deg3 = deg.reshape(NC, NP, 1)
    g1 = _tc_k1(x_p, W1, deg3)
    agg1 = _sc_aggregate(g1, src2, dst2, zeros_feat)
    g2 = _tc_k3(agg1, g1, deg3, b1.reshape(1, HID), W2)
    agg2 = _sc_aggregate(g2, src2, dst2, zeros_feat)
    probs = _tc_k4(agg2, g2, deg3, b2.reshape(1, HID), W3, b3.reshape(1, OUT))
    return probs[:N]


# speedup vs baseline: 28.0786x; 1.1104x over previous
import functools

import jax
import jax.numpy as jnp
from jax import lax
from jax.experimental import pallas as pl
from jax.experimental.pallas import tpu as pltpu
from jax.experimental.pallas import tpu_sc as plsc

N = 10000
E = 160000
D_IN = 256
HID = 32
OUT = 2

NC = 2
NS = 16
NW = NC * NS

NP = 10240
C = 128
CHUNKS_W = 40
EW = C * CHUNKS_W
EP = EW * NW
ROWS_S = NP // NS

_sc_mesh = plsc.VectorSubcoreMesh(
    core_axis_name="c", subcore_axis_name="s", num_cores=NC, num_subcores=NS
)
_sc_params = pltpu.CompilerParams(
    use_tc_tiling_on_sc=False, needs_layout_passes=False
)


def _rsqrt16(d):
    i = plsc.bitcast(d, jnp.int32)
    i = jnp.int32(0x5F3759DF) - lax.shift_right_logical(i, 1)
    y = plsc.bitcast(i, jnp.float32)
    for _ in range(3):
        y = y * (1.5 - 0.5 * d * y * y)
    return y


@functools.partial(
    pl.kernel,
    out_type=jax.ShapeDtypeStruct((NC, NP), jnp.float32),
    mesh=_sc_mesh,
    scratch_types=[
        pltpu.VMEM((CHUNKS_W, C), jnp.int32),
        pltpu.VMEM((C,), jnp.float32),
        pltpu.VMEM_SHARED((NP,), jnp.float32),
    ],
    compiler_params=_sc_params,
)
def _sc_degree(dst_hbm, zeros_hbm, out_hbm, dst_v, ones_v, deg_sh):
    c = lax.axis_index("c")
    s = lax.axis_index("s")
    w = c * NS + s
    pltpu.sync_copy(zeros_hbm.at[pl.ds(s * ROWS_S, ROWS_S)],
                    deg_sh.at[pl.ds(s * ROWS_S, ROWS_S)])
    pltpu.sync_copy(dst_hbm.at[pl.ds(w * CHUNKS_W, CHUNKS_W)], dst_v)
    for i in range(C // 16):
        ones_v[pl.ds(i * 16, 16)] = jnp.ones((16,), jnp.float32)
    plsc.subcore_barrier()

    @pl.loop(0, CHUNKS_W)
    def _(j):
        pltpu.sync_copy(ones_v, deg_sh.at[dst_v.at[j]], add=True)

    plsc.subcore_barrier()
    pltpu.sync_copy(deg_sh.at[pl.ds(s * ROWS_S, ROWS_S)],
                    out_hbm.at[c, pl.ds(s * ROWS_S, ROWS_S)])


def _scale_rows(buf_v, norm_v):
    @pl.loop(0, ROWS_S // 16)
    def _(i):
        r0 = i * 16
        n16 = norm_v[pl.ds(r0, 16)]
        for b in range(16):
            nv = jnp.full((16,), n16[b], jnp.float32)
            buf_v[r0 + b, pl.ds(0, 16)] = buf_v[r0 + b, pl.ds(0, 16)] * nv
            buf_v[r0 + b, pl.ds(16, 16)] = buf_v[r0 + b, pl.ds(16, 16)] * nv


def _agg_common(src_v, dst_v, rows_v, g_sh, agg_sh, sem):
    @pl.loop(0, CHUNKS_W)
    def _(j):
        pltpu.async_copy(g_sh.at[src_v.at[j]], rows_v, sem).wait()
        pltpu.sync_copy(rows_v, agg_sh.at[dst_v.at[j]], add=True)


def _emit_y(c, row0, norm_v, abuf_v, gbuf_v, agg_sh, y_hbm):
    cf = jnp.where(c == 1, 1.0, 0.0).astype(jnp.float32)
    cv = jnp.full((16,), cf, jnp.float32)
    pltpu.sync_copy(agg_sh.at[pl.ds(row0, ROWS_S)], abuf_v)

    @pl.loop(0, ROWS_S // 16)
    def _(i):
        r0 = i * 16
        n16 = norm_v[pl.ds(r0, 16)]
        for b in range(16):
            nv = jnp.full((16,), n16[b], jnp.float32)
            a0 = abuf_v[r0 + b, pl.ds(0, 16)] + gbuf_v[r0 + b, pl.ds(0, 16)] * cv
            a1 = abuf_v[r0 + b, pl.ds(16, 16)] + gbuf_v[r0 + b, pl.ds(16, 16)] * cv
            abuf_v[r0 + b, pl.ds(0, 16)] = a0 * nv
            abuf_v[r0 + b, pl.ds(16, 16)] = a1 * nv

    pltpu.sync_copy(abuf_v, y_hbm.at[c, pl.ds(row0, ROWS_S)])


_agg_scratch = [
    pltpu.VMEM((CHUNKS_W, C), jnp.int32),
    pltpu.VMEM((CHUNKS_W, C), jnp.int32),
    pltpu.VMEM((C, HID), jnp.float32),
    pltpu.VMEM((ROWS_S,), jnp.float32),
    pltpu.VMEM((ROWS_S, HID), jnp.float32),
    pltpu.VMEM((ROWS_S, HID), jnp.float32),
    pltpu.VMEM_SHARED((NP, HID), jnp.float32),
    pltpu.VMEM_SHARED((NP, HID), jnp.float32),
    pltpu.SemaphoreType.DMA,
]


@functools.partial(
    pl.kernel,
    out_type=(jax.ShapeDtypeStruct((NC, NP, HID), jnp.float32),
              jax.ShapeDtypeStruct((NP,), jnp.float32)),
    mesh=_sc_mesh,
    scratch_types=[pltpu.VMEM((NC, ROWS_S), jnp.float32)] + _agg_scratch,
    compiler_params=_sc_params,
)
def _sc_agg1(h_hbm, deg_hbm, src_hbm, dst_hbm, zeros_hbm,
             y_hbm, norm_hbm,
             deg_v, src_v, dst_v, rows_v, norm_v, gbuf_v, abuf_v,
             g_sh, agg_sh, sem):
    c = lax.axis_index("c")
    s = lax.axis_index("s")
    w = c * NS + s
    row0 = s * ROWS_S
    pltpu.sync_copy(deg_hbm.at[:, pl.ds(row0, ROWS_S)], deg_v)
    pltpu.sync_copy(h_hbm.at[pl.ds(row0, ROWS_S)], gbuf_v)
    pltpu.sync_copy(zeros_hbm.at[pl.ds(row0, ROWS_S)],
                    agg_sh.at[pl.ds(row0, ROWS_S)])
    pltpu.sync_copy(src_hbm.at[pl.ds(w * CHUNKS_W, CHUNKS_W)], src_v)
    pltpu.sync_copy(dst_hbm.at[pl.ds(w * CHUNKS_W, CHUNKS_W)], dst_v)

    @pl.loop(0, ROWS_S // 16)
    def _(i):
        d = deg_v[0, pl.ds(i * 16, 16)] + deg_v[1, pl.ds(i * 16, 16)] + 1.0
        norm_v[pl.ds(i * 16, 16)] = _rsqrt16(d)

    _scale_rows(gbuf_v, norm_v)
    pltpu.sync_copy(gbuf_v, g_sh.at[pl.ds(row0, ROWS_S)])

    @pl.when(c == 0)
    def _():
        pltpu.sync_copy(norm_v, norm_hbm.at[pl.ds(row0, ROWS_S)])

    plsc.subcore_barrier()
    _agg_common(src_v, dst_v, rows_v, g_sh, agg_sh, sem)
    plsc.subcore_barrier()
    _emit_y(c, row0, norm_v, abuf_v, gbuf_v, agg_sh, y_hbm)


@functools.partial(
    pl.kernel,
    out_type=jax.ShapeDtypeStruct((NC, NP, HID), jnp.float32),
    mesh=_sc_mesh,
    scratch_types=_agg_scratch,
    compiler_params=_sc_params,
)
def _sc_agg2(h_hbm, norm_in_hbm, src_hbm, dst_hbm, zeros_hbm,
             y_hbm,
             src_v, dst_v, rows_v, norm_v, gbuf_v, abuf_v,
             g_sh, agg_sh, sem):
    c = lax.axis_index("c")
    s = lax.axis_index("s")
    w = c * NS + s
    row0 = s * ROWS_S
    pltpu.sync_copy(norm_in_hbm.at[pl.ds(row0, ROWS_S)], norm_v)
    pltpu.sync_copy(h_hbm.at[pl.ds(row0, ROWS_S)], gbuf_v)
    pltpu.sync_copy(zeros_hbm.at[pl.ds(row0, ROWS_S)],
                    agg_sh.at[pl.ds(row0, ROWS_S)])
    pltpu.sync_copy(src_hbm.at[pl.ds(w * CHUNKS_W, CHUNKS_W)], src_v)
    pltpu.sync_copy(dst_hbm.at[pl.ds(w * CHUNKS_W, CHUNKS_W)], dst_v)
    _scale_rows(gbuf_v, norm_v)
    pltpu.sync_copy(gbuf_v, g_sh.at[pl.ds(row0, ROWS_S)])
    plsc.subcore_barrier()
    _agg_common(src_v, dst_v, rows_v, g_sh, agg_sh, sem)
    plsc.subcore_barrier()
    _emit_y(c, row0, norm_v, abuf_v, gbuf_v, agg_sh, y_hbm)


_BLK = 2560


def _k1_body(x_ref, w1_ref, h1_ref):
    h1_ref[...] = jnp.dot(x_ref[...], w1_ref[...],
                          preferred_element_type=jnp.float32)


def _k3_body(y_ref, b1_ref, w2_ref, h2_ref):
    h = jnp.maximum(y_ref[0] + y_ref[1] + b1_ref[...], 0.0)
    h2_ref[...] = jnp.dot(h, w2_ref[...], preferred_element_type=jnp.float32)


def _k4_body(y_ref, b2_ref, w3_ref, b3_ref, out_ref):
    h = jnp.maximum(y_ref[0] + y_ref[1] + b2_ref[...], 0.0)
    logits = jnp.dot(h, w3_ref[...],
                     preferred_element_type=jnp.float32) + b3_ref[...]
    m = jnp.max(logits, axis=-1, keepdims=True)
    e = jnp.exp(logits - m)
    out_ref[...] = e / jnp.sum(e, axis=-1, keepdims=True)


def _tc_k1(x, W1):
    return pl.pallas_call(
        _k1_body,
        grid=(NP // _BLK,),
        in_specs=[
            pl.BlockSpec((_BLK, D_IN), lambda i: (i, 0)),
            pl.BlockSpec((D_IN, HID), lambda i: (0, 0)),
        ],
        out_specs=pl.BlockSpec((_BLK, HID), lambda i: (i, 0)),
        out_shape=jax.ShapeDtypeStruct((NP, HID), jnp.float32),
    )(x, W1)


def _tc_k3(y, b1, W2):
    return pl.pallas_call(
        _k3_body,
        grid=(NP // _BLK,),
        in_specs=[
            pl.BlockSpec((NC, _BLK, HID), lambda i: (0, i, 0)),
            pl.BlockSpec((1, HID), lambda i: (0, 0)),
            pl.BlockSpec((HID, HID), lambda i: (0, 0)),
        ],
        out_specs=pl.BlockSpec((_BLK, HID), lambda i: (i, 0)),
        out_shape=jax.ShapeDtypeStruct((NP, HID), jnp.float32),
    )(y, b1, W2)


def _tc_k4(y, b2, W3, b3):
    return pl.pallas_call(
        _k4_body,
        grid=(NP // _BLK,),
        in_specs=[
            pl.BlockSpec((NC, _BLK, HID), lambda i: (0, i, 0)),
            pl.BlockSpec((1, HID), lambda i: (0, 0)),
            pl.BlockSpec((HID, OUT), lambda i: (0, 0)),
            pl.BlockSpec((1, OUT), lambda i: (0, 0)),
        ],
        out_specs=pl.BlockSpec((_BLK, OUT), lambda i: (i, 0)),
        out_shape=jax.ShapeDtypeStruct((NP, OUT), jnp.float32),
    )(y, b2, W3, b3)


@jax.jit
def kernel(x, edge_index, W1, b1, W2, b2, W3, b3):
    x_p = jnp.zeros((NP, D_IN), jnp.float32).at[:N].set(x)
    pad = jnp.full((2, EP - E), N, jnp.int32)
    ei = jnp.concatenate([edge_index, pad], axis=1)
    src2 = ei[0].reshape(NW * CHUNKS_W, C)
    dst2 = ei[1].reshape(NW * CHUNKS_W, C)
    zeros_col = jnp.zeros((NP,), jnp.float32)
    zeros_feat = jnp.zeros((NP, HID), jnp.float32)

    deg = _sc_degree(dst2, zeros_col)
    h1 = _tc_k1(x_p, W1)
    y1, norm = _sc_agg1(h1, deg, src2, dst2, zeros_feat)
    h2 = _tc_k3(y1, b1.reshape(1, HID), W2)
    y2 = _sc_agg2(h2, norm, src2, dst2, zeros_feat)
    probs = _tc_k4(y2, b2.reshape(1, HID), W3, b3.reshape(1, OUT))
    return probs[:N]


# speedup vs baseline: 30.0994x; 1.0720x over previous
import functools

import jax
import jax.numpy as jnp
from jax import lax
from jax.experimental import pallas as pl
from jax.experimental.pallas import tpu as pltpu
from jax.experimental.pallas import tpu_sc as plsc

N = 10000
E = 160000
D_IN = 256
HID = 32
OUT = 2

NC = 2
NS = 16
NW = NC * NS

NP = 10240
C = 128
CHUNKS_W = 40
EW = C * CHUNKS_W
EP = EW * NW
ROWS_S = NP // NS

_sc_mesh = plsc.VectorSubcoreMesh(
    core_axis_name="c", subcore_axis_name="s", num_cores=NC, num_subcores=NS
)
_sc_params = pltpu.CompilerParams(
    use_tc_tiling_on_sc=False, needs_layout_passes=False
)


def _rsqrt16(d):
    i = plsc.bitcast(d, jnp.int32)
    i = jnp.int32(0x5F3759DF) - lax.shift_right_logical(i, 1)
    y = plsc.bitcast(i, jnp.float32)
    for _ in range(3):
        y = y * (1.5 - 0.5 * d * y * y)
    return y


@functools.partial(
    pl.kernel,
    out_type=jax.ShapeDtypeStruct((NC, NP), jnp.float32),
    mesh=_sc_mesh,
    scratch_types=[
        pltpu.VMEM((CHUNKS_W, C), jnp.int32),
        pltpu.VMEM((C,), jnp.float32),
        pltpu.VMEM_SHARED((NP,), jnp.float32),
    ],
    compiler_params=_sc_params,
)
def _sc_degree(dst_hbm, zeros_hbm, out_hbm, dst_v, ones_v, deg_sh):
    c = lax.axis_index("c")
    s = lax.axis_index("s")
    w = c * NS + s
    pltpu.sync_copy(zeros_hbm.at[pl.ds(s * ROWS_S, ROWS_S)],
                    deg_sh.at[pl.ds(s * ROWS_S, ROWS_S)])
    pltpu.sync_copy(dst_hbm.at[pl.ds(w * CHUNKS_W, CHUNKS_W)], dst_v)
    for i in range(C // 16):
        ones_v[pl.ds(i * 16, 16)] = jnp.ones((16,), jnp.float32)
    plsc.subcore_barrier()

    @pl.loop(0, CHUNKS_W)
    def _(j):
        pltpu.sync_copy(ones_v, deg_sh.at[dst_v.at[j]], add=True)

    plsc.subcore_barrier()
    pltpu.sync_copy(deg_sh.at[pl.ds(s * ROWS_S, ROWS_S)],
                    out_hbm.at[c, pl.ds(s * ROWS_S, ROWS_S)])


def _scale_rows(buf_v, norm_v):
    @pl.loop(0, ROWS_S // 16)
    def _(i):
        r0 = i * 16
        n16 = norm_v[pl.ds(r0, 16)]
        for b in range(16):
            nv = jnp.full((16,), n16[b], jnp.float32)
            buf_v[r0 + b, pl.ds(0, 16)] = buf_v[r0 + b, pl.ds(0, 16)] * nv
            buf_v[r0 + b, pl.ds(16, 16)] = buf_v[r0 + b, pl.ds(16, 16)] * nv


def _agg_common(src_v, dst_v, rows_a, rows_b, g_sh, agg_sh, ga, gb, sa, sb):
    HALF = CHUNKS_W // 2

    pltpu.async_copy(g_sh.at[src_v.at[0]], rows_a, ga)

    @pl.loop(0, HALF)
    def _(jj):
        j = 2 * jj
        pltpu.make_async_copy(g_sh.at[src_v.at[j]], rows_a, ga).wait()
        pltpu.async_copy(rows_a, agg_sh.at[dst_v.at[j]], sa, add=True)

        @pl.when(jj > 0)
        def _():
            pltpu.make_async_copy(rows_b, agg_sh.at[dst_v.at[j]], sb).wait()

        pltpu.async_copy(g_sh.at[src_v.at[j + 1]], rows_b, gb)
        pltpu.make_async_copy(g_sh.at[src_v.at[j + 1]], rows_b, gb).wait()
        pltpu.async_copy(rows_b, agg_sh.at[dst_v.at[j + 1]], sb, add=True)

        @pl.when(jj < HALF - 1)
        def _():
            pltpu.make_async_copy(rows_a, agg_sh.at[dst_v.at[j]], sa).wait()
            pltpu.async_copy(g_sh.at[src_v.at[j + 2]], rows_a, ga)

    pltpu.make_async_copy(rows_a, agg_sh.at[dst_v.at[0]], sa).wait()
    pltpu.make_async_copy(rows_b, agg_sh.at[dst_v.at[0]], sb).wait()


def _emit_y(c, row0, norm_v, abuf_v, gbuf_v, agg_sh, y_hbm):
    cf = jnp.where(c == 1, 1.0, 0.0).astype(jnp.float32)
    cv = jnp.full((16,), cf, jnp.float32)
    pltpu.sync_copy(agg_sh.at[pl.ds(row0, ROWS_S)], abuf_v)

    @pl.loop(0, ROWS_S // 16)
    def _(i):
        r0 = i * 16
        n16 = norm_v[pl.ds(r0, 16)]
        for b in range(16):
            nv = jnp.full((16,), n16[b], jnp.float32)
            a0 = abuf_v[r0 + b, pl.ds(0, 16)] + gbuf_v[r0 + b, pl.ds(0, 16)] * cv
            a1 = abuf_v[r0 + b, pl.ds(16, 16)] + gbuf_v[r0 + b, pl.ds(16, 16)] * cv
            abuf_v[r0 + b, pl.ds(0, 16)] = a0 * nv
            abuf_v[r0 + b, pl.ds(16, 16)] = a1 * nv

    pltpu.sync_copy(abuf_v, y_hbm.at[c, pl.ds(row0, ROWS_S)])


_agg_scratch = [
    pltpu.VMEM((CHUNKS_W, C), jnp.int32),
    pltpu.VMEM((CHUNKS_W, C), jnp.int32),
    pltpu.VMEM((C, HID), jnp.float32),
    pltpu.VMEM((C, HID), jnp.float32),
    pltpu.VMEM((ROWS_S,), jnp.float32),
    pltpu.VMEM((ROWS_S, HID), jnp.float32),
    pltpu.VMEM((ROWS_S, HID), jnp.float32),
    pltpu.VMEM_SHARED((NP, HID), jnp.float32),
    pltpu.VMEM_SHARED((NP, HID), jnp.float32),
    pltpu.SemaphoreType.DMA,
    pltpu.SemaphoreType.DMA,
    pltpu.SemaphoreType.DMA,
    pltpu.SemaphoreType.DMA,
]


@functools.partial(
    pl.kernel,
    out_type=(jax.ShapeDtypeStruct((NC, NP, HID), jnp.float32),
              jax.ShapeDtypeStruct((NP,), jnp.float32)),
    mesh=_sc_mesh,
    scratch_types=[pltpu.VMEM((NC, ROWS_S), jnp.float32)] + _agg_scratch,
    compiler_params=_sc_params,
)
def _sc_agg1(h_hbm, deg_hbm, src_hbm, dst_hbm, zeros_hbm,
             y_hbm, norm_hbm,
             deg_v, src_v, dst_v, rows_a, rows_b, norm_v, gbuf_v, abuf_v,
             g_sh, agg_sh, ga, gb, sa, sb):
    c = lax.axis_index("c")
    s = lax.axis_index("s")
    w = c * NS + s
    row0 = s * ROWS_S
    pltpu.sync_copy(deg_hbm.at[:, pl.ds(row0, ROWS_S)], deg_v)
    pltpu.sync_copy(h_hbm.at[pl.ds(row0, ROWS_S)], gbuf_v)
    pltpu.sync_copy(zeros_hbm.at[pl.ds(row0, ROWS_S)],
                    agg_sh.at[pl.ds(row0, ROWS_S)])
    pltpu.sync_copy(src_hbm.at[pl.ds(w * CHUNKS_W, CHUNKS_W)], src_v)
    pltpu.sync_copy(dst_hbm.at[pl.ds(w * CHUNKS_W, CHUNKS_W)], dst_v)

    @pl.loop(0, ROWS_S // 16)
    def _(i):
        d = deg_v[0, pl.ds(i * 16, 16)] + deg_v[1, pl.ds(i * 16, 16)] + 1.0
        norm_v[pl.ds(i * 16, 16)] = _rsqrt16(d)

    _scale_rows(gbuf_v, norm_v)
    pltpu.sync_copy(gbuf_v, g_sh.at[pl.ds(row0, ROWS_S)])

    @pl.when(c == 0)
    def _():
        pltpu.sync_copy(norm_v, norm_hbm.at[pl.ds(row0, ROWS_S)])

    plsc.subcore_barrier()
    _agg_common(src_v, dst_v, rows_a, rows_b, g_sh, agg_sh, ga, gb, sa, sb)
    plsc.subcore_barrier()
    _emit_y(c, row0, norm_v, abuf_v, gbuf_v, agg_sh, y_hbm)


@functools.partial(
    pl.kernel,
    out_type=jax.ShapeDtypeStruct((NC, NP, HID), jnp.float32),
    mesh=_sc_mesh,
    scratch_types=_agg_scratch,
    compiler_params=_sc_params,
)
def _sc_agg2(h_hbm, norm_in_hbm, src_hbm, dst_hbm, zeros_hbm,
             y_hbm,
             src_v, dst_v, rows_a, rows_b, norm_v, gbuf_v, abuf_v,
             g_sh, agg_sh, ga, gb, sa, sb):
    c = lax.axis_index("c")
    s = lax.axis_index("s")
    w = c * NS + s
    row0 = s * ROWS_S
    pltpu.sync_copy(norm_in_hbm.at[pl.ds(row0, ROWS_S)], norm_v)
    pltpu.sync_copy(h_hbm.at[pl.ds(row0, ROWS_S)], gbuf_v)
    pltpu.sync_copy(zeros_hbm.at[pl.ds(row0, ROWS_S)],
                    agg_sh.at[pl.ds(row0, ROWS_S)])
    pltpu.sync_copy(src_hbm.at[pl.ds(w * CHUNKS_W, CHUNKS_W)], src_v)
    pltpu.sync_copy(dst_hbm.at[pl.ds(w * CHUNKS_W, CHUNKS_W)], dst_v)
    _scale_rows(gbuf_v, norm_v)
    pltpu.sync_copy(gbuf_v, g_sh.at[pl.ds(row0, ROWS_S)])
    plsc.subcore_barrier()
    _agg_common(src_v, dst_v, rows_a, rows_b, g_sh, agg_sh, ga, gb, sa, sb)
    plsc.subcore_barrier()
    _emit_y(c, row0, norm_v, abuf_v, gbuf_v, agg_sh, y_hbm)


_BLK = 2560


def _k1_body(x_ref, w1_ref, h1_ref):
    h1_ref[...] = jnp.dot(x_ref[...], w1_ref[...],
                          preferred_element_type=jnp.float32)


def _k3_body(y_ref, b1_ref, w2_ref, h2_ref):
    h = jnp.maximum(y_ref[0] + y_ref[1] + b1_ref[...], 0.0)
    h2_ref[...] = jnp.dot(h, w2_ref[...], preferred_element_type=jnp.float32)


def _k4_body(y_ref, b2_ref, w3_ref, b3_ref, out_ref):
    h = jnp.maximum(y_ref[0] + y_ref[1] + b2_ref[...], 0.0)
    logits = jnp.dot(h, w3_ref[...],
                     preferred_element_type=jnp.float32) + b3_ref[...]
    m = jnp.max(logits, axis=-1, keepdims=True)
    e = jnp.exp(logits - m)
    out_ref[...] = e / jnp.sum(e, axis=-1, keepdims=True)


def _tc_k1(x, W1):
    return pl.pallas_call(
        _k1_body,
        grid=(NP // _BLK,),
        in_specs=[
            pl.BlockSpec((_BLK, D_IN), lambda i: (i, 0)),
            pl.BlockSpec((D_IN, HID), lambda i: (0, 0)),
        ],
        out_specs=pl.BlockSpec((_BLK, HID), lambda i: (i, 0)),
        out_shape=jax.ShapeDtypeStruct((NP, HID), jnp.float32),
    )(x, W1)


def _tc_k3(y, b1, W2):
    return pl.pallas_call(
        _k3_body,
        grid=(NP // _BLK,),
        in_specs=[
            pl.BlockSpec((NC, _BLK, HID), lambda i: (0, i, 0)),
            pl.BlockSpec((1, HID), lambda i: (0, 0)),
            pl.BlockSpec((HID, HID), lambda i: (0, 0)),
        ],
        out_specs=pl.BlockSpec((_BLK, HID), lambda i: (i, 0)),
        out_shape=jax.ShapeDtypeStruct((NP, HID), jnp.float32),
    )(y, b1, W2)


def _tc_k4(y, b2, W3, b3):
    return pl.pallas_call(
        _k4_body,
        grid=(NP // _BLK,),
        in_specs=[
            pl.BlockSpec((NC, _BLK, HID), lambda i: (0, i, 0)),
            pl.BlockSpec((1, HID), lambda i: (0, 0)),
            pl.BlockSpec((HID, OUT), lambda i: (0, 0)),
            pl.BlockSpec((1, OUT), lambda i: (0, 0)),
        ],
        out_specs=pl.BlockSpec((_BLK, OUT), lambda i: (i, 0)),
        out_shape=jax.ShapeDtypeStruct((NP, OUT), jnp.float32),
    )(y, b2, W3, b3)


@jax.jit
def kernel(x, edge_index, W1, b1, W2, b2, W3, b3):
    x_p = jnp.zeros((NP, D_IN), jnp.float32).at[:N].set(x)
    pad = jnp.full((2, EP - E), N, jnp.int32)
    ei = jnp.concatenate([edge_index, pad], axis=1)
    src2 = ei[0].reshape(NW * CHUNKS_W, C)
    dst2 = ei[1].reshape(NW * CHUNKS_W, C)
    zeros_col = jnp.zeros((NP,), jnp.float32)
    zeros_feat = jnp.zeros((NP, HID), jnp.float32)

    deg = _sc_degree(dst2, zeros_col)
    h1 = _tc_k1(x_p, W1)
    y1, norm = _sc_agg1(h1, deg, src2, dst2, zeros_feat)
    h2 = _tc_k3(y1, b1.reshape(1, HID), W2)
    y2 = _sc_agg2(h2, norm, src2, dst2, zeros_feat)
    probs = _tc_k4(y2, b2.reshape(1, HID), W3, b3.reshape(1, OUT))
    return probs[:N]
